# baseline (speedup 1.0000x reference)
import functools

import jax
import jax.numpy as jnp
from jax import lax
from jax.experimental import pallas as pl
from jax.experimental.pallas import tpu as pltpu
from jax.experimental.pallas import tpu_sc as plsc

BATCH = 16384
STUDENT_N = 1000000
EXER_N = 100000
NUM_CORES = 2
NUM_SUBCORES = 16
LANES = 16
NUM_WORKERS = NUM_CORES * NUM_SUBCORES
B_PER_W = BATCH // NUM_WORKERS


def _flatten_body(s_ref, k_ref, d_ref, so_ref, ko_ref, do_ref, sem):
    cs = pltpu.async_copy(s_ref.at[0, :], so_ref, sem)
    ck = pltpu.async_copy(k_ref.at[0, :], ko_ref, sem)
    cd = pltpu.async_copy(d_ref.at[0, :], do_ref, sem)
    cs.wait()
    ck.wait()
    cd.wait()


def _tc_flatten(s, k, d):
    return pl.pallas_call(
        _flatten_body,
        out_shape=(
            jax.ShapeDtypeStruct((STUDENT_N,), jnp.float32),
            jax.ShapeDtypeStruct((EXER_N,), jnp.float32),
            jax.ShapeDtypeStruct((EXER_N,), jnp.float32),
        ),
        in_specs=[
            pl.BlockSpec(memory_space=pl.ANY),
            pl.BlockSpec(memory_space=pl.ANY),
            pl.BlockSpec(memory_space=pl.ANY),
        ],
        out_specs=(
            pl.BlockSpec(memory_space=pl.ANY),
            pl.BlockSpec(memory_space=pl.ANY),
            pl.BlockSpec(memory_space=pl.ANY),
        ),
        scratch_shapes=[pltpu.SemaphoreType.DMA],
    )(s.T, k.T, d.T)


def _build_sc_kernel():
    mesh = plsc.VectorSubcoreMesh(core_axis_name="c", subcore_axis_name="s")

    @functools.partial(
        pl.kernel,
        mesh=mesh,
        out_type=jax.ShapeDtypeStruct((BATCH,), jnp.float32),
        scratch_types=[
            pltpu.VMEM((B_PER_W,), jnp.int32),
            pltpu.VMEM((B_PER_W,), jnp.int32),
            pltpu.VMEM((B_PER_W,), jnp.float32),
            pltpu.VMEM((B_PER_W,), jnp.float32),
            pltpu.VMEM((B_PER_W,), jnp.float32),
            pltpu.VMEM((B_PER_W,), jnp.float32),
            pltpu.SemaphoreType.DMA,
            pltpu.SemaphoreType.DMA,
        ],
    )
    def sc_kernel(stu_id_hbm, exer_id_hbm, stu_emb_hbm, kdiff_hbm, edisc_hbm,
                  out_hbm, sidx_v, eidx_v, s_v, k_v, d_v, o_v, sem, isem):
        wid = lax.axis_index("s") * NUM_CORES + lax.axis_index("c")
        base = wid * B_PER_W
        ci_e = pltpu.async_copy(exer_id_hbm.at[pl.ds(base, B_PER_W)], eidx_v, isem)
        ci_s = pltpu.async_copy(stu_id_hbm.at[pl.ds(base, B_PER_W)], sidx_v, isem)
        ci_e.wait()
        c_k = pltpu.async_copy(kdiff_hbm.at[eidx_v], k_v, sem)
        c_d = pltpu.async_copy(edisc_hbm.at[eidx_v], d_v, sem)
        ci_s.wait()
        c_s = pltpu.async_copy(stu_emb_hbm.at[sidx_v], s_v, sem)
        c_k.wait()
        c_d.wait()
        c_s.wait()

        def body(i, carry):
            sl = pl.ds(i * LANES, LANES)
            es = jnp.exp(-s_v[sl])
            ek = jnp.exp(-k_v[sl])
            ed = jnp.exp(-d_v[sl])
            t = (10.0 * (ek - es)) / ((1.0 + es) * ((1.0 + ek) * (1.0 + ed)))
            o_v[sl] = 1.0 / (1.0 + jnp.exp(-t))
            return carry

        lax.fori_loop(0, B_PER_W // LANES, body, 0)
        pltpu.sync_copy(o_v, out_hbm.at[pl.ds(base, B_PER_W)])

    return sc_kernel


_SC_KERNEL = _build_sc_kernel()


@jax.jit
def kernel(stu_id, exer_id, student_emb, k_difficulty, e_discrimination):
    s_flat, k_flat, d_flat = _tc_flatten(
        student_emb, k_difficulty, e_discrimination)
    out = _SC_KERNEL(
        stu_id.astype(jnp.int32),
        exer_id.astype(jnp.int32),
        s_flat,
        k_flat,
        d_flat,
    )
    return out.reshape(BATCH, 1)


# speedup vs baseline: 6.1747x; 6.1747x over previous
import functools
import jax, jax.numpy as jnp
from jax import lax
from jax.experimental import pallas as pl
from jax.experimental.pallas import tpu as pltpu, tpu_sc as plsc

BATCH = 16384
SN = 1000000
EN = 100000
NC, NS, L = 2, 16, 16
NW = NC * NS
BW = BATCH // NW

S_STRIPE = 62464
S_LAST_OFF = 15 * S_STRIPE
S_LAST_MAIN = 62976
S_MAIN = 999936
E_STRIPE = 6144
E_LAST_OFF = 15 * E_STRIPE
E_LAST_MAIN = 7808
E_MAIN = 99968

mesh = plsc.VectorSubcoreMesh(core_axis_name="c", subcore_axis_name="s")


@functools.partial(
    pl.kernel, mesh=mesh,
    out_type=jax.ShapeDtypeStruct((BATCH,), jnp.float32),
    scratch_types=[
        pltpu.VMEM_SHARED((SN + 64,), jnp.float32),
        pltpu.VMEM_SHARED((EN + 96,), jnp.float32),
        pltpu.VMEM_SHARED((EN + 96,), jnp.float32),
        pltpu.VMEM((BW,), jnp.int32),
        pltpu.VMEM((BW,), jnp.int32),
        pltpu.VMEM((BW,), jnp.float32),
        pltpu.VMEM((BW,), jnp.float32),
        pltpu.VMEM((BW,), jnp.float32),
        pltpu.VMEM((BW,), jnp.float32),
        pltpu.SemaphoreType.DMA,
        pltpu.SemaphoreType.DMA,
    ],
)
def _k(stu_id_hbm, exer_id_hbm, sT_hbm, kT_hbm, dT_hbm, st_hbm, kt_hbm, dt_hbm, out_hbm,
       sh_s, sh_k, sh_d, sidx_v, eidx_v, s_v, k_v, d_v, o_v, sem, isem):
    sid = lax.axis_index("s")
    wid = sid * NC + lax.axis_index("c")
    base = wid * BW
    ci_e = pltpu.async_copy(exer_id_hbm.at[pl.ds(base, BW)], eidx_v, isem)
    ci_s = pltpu.async_copy(stu_id_hbm.at[pl.ds(base, BW)], sidx_v, isem)

    def stage(src, dst, off, n):
        off = pl.multiple_of(off, 128)
        return pltpu.async_copy(
            src.at[0, pl.ds(off, n)], dst.at[pl.ds(off, n)], sem)

    @pl.when(sid < NS - 1)
    def _():
        c1 = stage(sT_hbm, sh_s, sid * S_STRIPE, S_STRIPE)
        c2 = stage(kT_hbm, sh_k, sid * E_STRIPE, E_STRIPE)
        c3 = stage(dT_hbm, sh_d, sid * E_STRIPE, E_STRIPE)
        c1.wait(); c2.wait(); c3.wait()

    @pl.when(sid == NS - 1)
    def _():
        c1 = stage(sT_hbm, sh_s, S_LAST_OFF, S_LAST_MAIN)
        c2 = stage(kT_hbm, sh_k, E_LAST_OFF, E_LAST_MAIN)
        c3 = stage(dT_hbm, sh_d, E_LAST_OFF, E_LAST_MAIN)
        c4 = pltpu.async_copy(st_hbm.at[0, pl.ds(0, 128)],
                              sh_s.at[pl.ds(S_MAIN, 128)], sem)
        c5 = pltpu.async_copy(kt_hbm.at[0, pl.ds(0, 128)],
                              sh_k.at[pl.ds(E_MAIN, 128)], sem)
        c6 = pltpu.async_copy(dt_hbm.at[0, pl.ds(0, 128)],
                              sh_d.at[pl.ds(E_MAIN, 128)], sem)
        c1.wait(); c2.wait(); c3.wait(); c4.wait(); c5.wait(); c6.wait()

    plsc.subcore_barrier()

    ci_e.wait()
    c_k = pltpu.async_copy(sh_k.at[eidx_v], k_v, sem)
    c_d = pltpu.async_copy(sh_d.at[eidx_v], d_v, sem)
    ci_s.wait()
    c_s = pltpu.async_copy(sh_s.at[sidx_v], s_v, sem)
    c_k.wait()
    c_d.wait()
    c_s.wait()

    def body(i, carry):
        sl = pl.ds(i * L, L)
        es = jnp.exp(-s_v[sl])
        ek = jnp.exp(-k_v[sl])
        ed = jnp.exp(-d_v[sl])
        t = (10.0 * (ek - es)) / ((1.0 + es) * ((1.0 + ek) * (1.0 + ed)))
        o_v[sl] = 1.0 / (1.0 + jnp.exp(-t))
        return carry

    lax.fori_loop(0, BW // L, body, 0)
    pltpu.sync_copy(o_v, out_hbm.at[pl.ds(base, BW)])


@jax.jit
def kernel(stu_id, exer_id, student_emb, k_difficulty, e_discrimination):
    out = _k(
        stu_id.astype(jnp.int32),
        exer_id.astype(jnp.int32),
        student_emb.T,
        k_difficulty.T,
        e_discrimination.T,
        jnp.pad(student_emb[999936:, 0], (0, 64)).reshape(1, 128),
        jnp.pad(k_difficulty[99968:, 0], (0, 96)).reshape(1, 128),
        jnp.pad(e_discrimination[99968:, 0], (0, 96)).reshape(1, 128),
    )
    return out.reshape(BATCH, 1)


# speedup vs baseline: 6.1920x; 1.0028x over previous
import functools

import jax
import jax.numpy as jnp
from jax import lax
from jax.experimental import pallas as pl
from jax.experimental.pallas import tpu as pltpu
from jax.experimental.pallas import tpu_sc as plsc

BATCH = 16384
SN = 1000000
EN = 100000
NC, NS, L = 2, 16, 16
NW = NC * NS
BW = BATCH // NW

S_STRIPE = 62464
S_LAST_OFF = 15 * S_STRIPE
S_LAST_MAIN = 62976
S_MAIN = 999936
E_STRIPE = 6144
E_LAST_OFF = 15 * E_STRIPE
E_LAST_MAIN = 7808
E_MAIN = 99968

mesh = plsc.VectorSubcoreMesh(core_axis_name="c", subcore_axis_name="s")


@functools.partial(
    pl.kernel, mesh=mesh,
    out_type=jax.ShapeDtypeStruct((BATCH,), jnp.float32),
    scratch_types=[
        pltpu.VMEM_SHARED((SN + 64,), jnp.float32),
        pltpu.VMEM_SHARED((EN + 96,), jnp.float32),
        pltpu.VMEM_SHARED((EN + 96,), jnp.float32),
        pltpu.VMEM((BW,), jnp.int32),
        pltpu.VMEM((BW,), jnp.int32),
        pltpu.VMEM((BW,), jnp.float32),
        pltpu.VMEM((BW,), jnp.float32),
        pltpu.VMEM((BW,), jnp.float32),
        pltpu.VMEM((BW,), jnp.float32),
        pltpu.SemaphoreType.DMA,
        pltpu.SemaphoreType.DMA,
    ],
)
def _k(stu_id_hbm, exer_id_hbm, sT_hbm, kT_hbm, dT_hbm, tails_hbm, out_hbm,
       sh_s, sh_k, sh_d, sidx_v, eidx_v, s_v, k_v, d_v, o_v, sem, isem):
    sid = lax.axis_index("s")
    wid = sid * NC + lax.axis_index("c")
    base = wid * BW
    ci_e = pltpu.async_copy(exer_id_hbm.at[pl.ds(base, BW)], eidx_v, isem)
    ci_s = pltpu.async_copy(stu_id_hbm.at[pl.ds(base, BW)], sidx_v, isem)

    def stage(src, dst, off, n):
        off = pl.multiple_of(off, 128)
        return pltpu.async_copy(
            src.at[0, pl.ds(off, n)], dst.at[pl.ds(off, n)], sem)

    @pl.when(sid < NS - 1)
    def _():
        c_k = stage(kT_hbm, sh_k, sid * E_STRIPE, E_STRIPE)
        c_d = stage(dT_hbm, sh_d, sid * E_STRIPE, E_STRIPE)
        c_s = stage(sT_hbm, sh_s, sid * S_STRIPE, S_STRIPE)
        c_k.wait()
        c_d.wait()
        plsc.subcore_barrier()
        ci_e.wait()
        g_k = pltpu.async_copy(sh_k.at[eidx_v], k_v, isem)
        g_d = pltpu.async_copy(sh_d.at[eidx_v], d_v, isem)
        c_s.wait()
        plsc.subcore_barrier()
        ci_s.wait()
        g_s = pltpu.async_copy(sh_s.at[sidx_v], s_v, isem)
        g_k.wait()
        g_d.wait()
        g_s.wait()

    @pl.when(sid == NS - 1)
    def _():
        c_k = stage(kT_hbm, sh_k, E_LAST_OFF, E_LAST_MAIN)
        c_d = stage(dT_hbm, sh_d, E_LAST_OFF, E_LAST_MAIN)
        c_kt = pltpu.async_copy(tails_hbm.at[0, pl.ds(128, 128)],
                                sh_k.at[pl.ds(E_MAIN, 128)], sem)
        c_dt = pltpu.async_copy(tails_hbm.at[0, pl.ds(256, 128)],
                                sh_d.at[pl.ds(E_MAIN, 128)], sem)
        c_s = stage(sT_hbm, sh_s, S_LAST_OFF, S_LAST_MAIN)
        c_st = pltpu.async_copy(tails_hbm.at[0, pl.ds(0, 128)],
                                sh_s.at[pl.ds(S_MAIN, 128)], sem)
        c_k.wait()
        c_d.wait()
        c_kt.wait()
        c_dt.wait()
        plsc.subcore_barrier()
        ci_e.wait()
        g_k = pltpu.async_copy(sh_k.at[eidx_v], k_v, isem)
        g_d = pltpu.async_copy(sh_d.at[eidx_v], d_v, isem)
        c_s.wait()
        c_st.wait()
        plsc.subcore_barrier()
        ci_s.wait()
        g_s = pltpu.async_copy(sh_s.at[sidx_v], s_v, isem)
        g_k.wait()
        g_d.wait()
        g_s.wait()

    def body(i, carry):
        sl = pl.ds(i * L, L)
        es = jnp.exp(-s_v[sl])
        ek = jnp.exp(-k_v[sl])
        ed = jnp.exp(-d_v[sl])
        t = (10.0 * (ek - es)) / ((1.0 + es) * ((1.0 + ek) * (1.0 + ed)))
        o_v[sl] = 1.0 / (1.0 + jnp.exp(-t))
        return carry

    lax.fori_loop(0, BW // L, body, 0)
    pltpu.sync_copy(o_v, out_hbm.at[pl.ds(base, BW)])


@jax.jit
def kernel(stu_id, exer_id, student_emb, k_difficulty, e_discrimination):
    z96 = jnp.zeros((96,), jnp.float32)
    tails = jnp.concatenate([
        student_emb[S_MAIN:, 0], jnp.zeros((64,), jnp.float32),
        k_difficulty[E_MAIN:, 0], z96,
        e_discrimination[E_MAIN:, 0], z96,
    ]).reshape(1, 384)
    out = _k(
        stu_id.astype(jnp.int32),
        exer_id.astype(jnp.int32),
        student_emb.T,
        k_difficulty.T,
        e_discrimination.T,
        tails,
    )
    return out.reshape(BATCH, 1)
